# Initial kernel scaffold; baseline (speedup 1.0000x reference)
import jax
import jax.numpy as jnp
from jax.experimental import pallas as pl


def kernel(prev, x, w1, b1, gamma1, beta1, w2, b2, gamma2, beta2, w3, b3, gamma3, beta3):
    raise NotImplementedError("write your pallas kernel here")


# speedup vs baseline: 1.4474x; 1.4474x over previous
import functools

import jax
import jax.numpy as jnp
from jax.experimental import pallas as pl
from jax.experimental.pallas import tpu as pltpu

_CD = jnp.bfloat16
_NT = 3


def _compiler_params():
    return pltpu.CompilerParams(
        dimension_semantics=("parallel", "arbitrary"),
        vmem_limit_bytes=56 * 1024 * 1024)


def _up_vec(n, m):
    k = jnp.arange(m) // 2
    p = jnp.arange(m) % 2
    nb = jnp.where(p == 0, jnp.maximum(k - 1, 0), jnp.minimum(k + 1, n - 1))
    r = jnp.arange(n)[:, None]
    return 0.75 * (r == k[None, :]) + 0.25 * (r == nb[None, :])


def _upsample_matrix(h, w, wp):
    uv = _up_vec(h, 2 * h)
    uh = jnp.pad(_up_vec(w, 2 * w), ((0, 0), (1, 1)))
    b = uv[:, None, :, None] * uh[None, :, None, :]
    return b.reshape(h * w, 2 * h * wp)


def _conv_tile(xfull_ref, w_ref, y_ref, ps_ref, pss_ref, *, wp, lt):
    t = pl.program_id(1)
    f0 = t * lt
    win = xfull_ref[:, pl.ds(f0, lt + 256)]
    stk = jnp.concatenate(
        [win[:, ky * wp + kx:ky * wp + kx + lt]
         for ky in range(3) for kx in range(3)], axis=0)
    acc = jnp.dot(w_ref[...], stk, preferred_element_type=jnp.float32)
    y_ref[0] = acc.astype(y_ref.dtype)
    col = (jax.lax.broadcasted_iota(jnp.int32, (1, lt), 1) + f0) % wp
    am = jnp.where((col >= 1) & (col <= wp - 2), acc, 0.0)
    ps_ref[0, 0] = jnp.sum(am, axis=1, keepdims=True)
    pss_ref[0, 0] = jnp.sum(am * am, axis=1, keepdims=True)


def _conv1_kernel(x_ref, prev_ref, b_ref, w_ref, y_ref, ps_ref, pss_ref,
                  xfull_ref, *, wp, lt):
    @pl.when(pl.program_id(1) == 0)
    def _stage():
        cin, xl = xfull_ref.shape
        lf = prev_ref.shape[2]
        ax = jnp.dot(x_ref[0].astype(_CD), b_ref[...],
                     preferred_element_type=jnp.float32).astype(_CD)
        a = jnp.concatenate([ax, prev_ref[0]], axis=0)
        xfull_ref[...] = jnp.concatenate(
            [jnp.zeros((cin, wp + 1), _CD), a,
             jnp.zeros((cin, xl - wp - 1 - lf), _CD)], axis=1)

    _conv_tile(xfull_ref, w_ref, y_ref, ps_ref, pss_ref, wp=wp, lt=lt)


def _conv_mid_kernel(yin_ref, w_ref, sc_ref, sh_ref, y_ref, ps_ref, pss_ref,
                     xfull_ref, *, wp, lt):
    @pl.when(pl.program_id(1) == 0)
    def _stage():
        cin, xl = xfull_ref.shape
        lf = yin_ref.shape[2]
        v = yin_ref[0].astype(jnp.float32) * sc_ref[...] + sh_ref[...]
        v = jnp.maximum(v, 0.0)
        col = jax.lax.broadcasted_iota(jnp.int32, (1, lf), 1) % wp
        a = jnp.where((col >= 1) & (col <= wp - 2), v, 0.0).astype(_CD)
        xfull_ref[...] = jnp.concatenate(
            [jnp.zeros((cin, wp + 1), _CD), a,
             jnp.zeros((cin, xl - wp - 1 - lf), _CD)], axis=1)

    _conv_tile(xfull_ref, w_ref, y_ref, ps_ref, pss_ref, wp=wp, lt=lt)


def _epilogue_kernel(yin_ref, sc_ref, sh_ref, o_ref, *, wp):
    c, h2, w2 = o_ref.shape[1], o_ref.shape[2], o_ref.shape[3]
    v = jnp.maximum(yin_ref[0].astype(jnp.float32) * sc_ref[...] + sh_ref[...],
                    0.0)
    o_ref[0] = v.reshape(c, h2, wp)[:, :, 1:w2 + 1]


def _fold_bn(ps, pss, gamma, beta, count, eps=1e-5):
    cout = ps.shape[2]
    s = jnp.sum(ps.reshape(-1, cout), axis=0)
    ss = jnp.sum(pss.reshape(-1, cout), axis=0)
    mean = s / count
    var = jnp.maximum(ss / count - mean * mean, 0.0)
    scale = gamma / jnp.sqrt(var + eps)
    shift = beta - mean * scale
    return scale.reshape(-1, 1), shift.reshape(-1, 1)


def _stack_taps(w_oihw):
    cout, cin = w_oihw.shape[0], w_oihw.shape[1]
    return jnp.transpose(w_oihw, (0, 2, 3, 1)).reshape(cout, 9 * cin).astype(_CD)


def kernel(prev, x, w1, b1, gamma1, beta1, w2, b2, gamma2, beta2,
           w3, b3, gamma3, beta3):
    n, cx, h, w = x.shape
    cp = prev.shape[1]
    h2, w2s = 2 * h, 2 * w
    wp = w2s + 2
    lf = h2 * wp
    lt = lf // _NT
    count = n * h2 * w2s
    cin1 = cx + cp
    cout = w1.shape[0]
    xl = -(-lf // 128) * 128 + 256

    xf = x.reshape(n, cx, h * w)
    prevf = jnp.pad(prev, ((0, 0), (0, 0), (0, 0), (1, 1))
                    ).reshape(n, cp, lf).astype(_CD)
    bmat = _upsample_matrix(h, w, wp).astype(_CD)

    def conv_out_shape():
        return (jax.ShapeDtypeStruct((n, cout, lf), _CD),
                jax.ShapeDtypeStruct((n, _NT, cout, 1), jnp.float32),
                jax.ShapeDtypeStruct((n, _NT, cout, 1), jnp.float32))

    def conv_out_specs():
        return (pl.BlockSpec((1, cout, lt), lambda i, t: (i, 0, t)),
                pl.BlockSpec((1, 1, cout, 1), lambda i, t: (i, t, 0, 0)),
                pl.BlockSpec((1, 1, cout, 1), lambda i, t: (i, t, 0, 0)))

    y1, ps, pss = pl.pallas_call(
        functools.partial(_conv1_kernel, wp=wp, lt=lt),
        out_shape=conv_out_shape(),
        grid=(n, _NT),
        in_specs=[
            pl.BlockSpec((1, cx, h * w), lambda i, t: (i, 0, 0)),
            pl.BlockSpec((1, cp, lf), lambda i, t: (i, 0, 0)),
            pl.BlockSpec((h * w, lf), lambda i, t: (0, 0)),
            pl.BlockSpec((cout, 9 * cin1), lambda i, t: (0, 0)),
        ],
        out_specs=conv_out_specs(),
        scratch_shapes=[pltpu.VMEM((cin1, xl), _CD)],
        compiler_params=_compiler_params(),
    )(xf, prevf, bmat, _stack_taps(w1))

    yk = y1
    scale, shift = _fold_bn(ps, pss, gamma1, beta1, count)
    for wi, gi, bi in ((w2, gamma2, beta2), (w3, gamma3, beta3)):
        ci = wi.shape[1]
        yk, ps, pss = pl.pallas_call(
            functools.partial(_conv_mid_kernel, wp=wp, lt=lt),
            out_shape=conv_out_shape(),
            grid=(n, _NT),
            in_specs=[
                pl.BlockSpec((1, ci, lf), lambda i, t: (i, 0, 0)),
                pl.BlockSpec((cout, 9 * ci), lambda i, t: (0, 0)),
                pl.BlockSpec((ci, 1), lambda i, t: (0, 0)),
                pl.BlockSpec((ci, 1), lambda i, t: (0, 0)),
            ],
            out_specs=conv_out_specs(),
            scratch_shapes=[pltpu.VMEM((ci, xl), _CD)],
            compiler_params=_compiler_params(),
        )(yk, _stack_taps(wi), scale, shift)
        scale, shift = _fold_bn(ps, pss, gi, bi, count)

    out = pl.pallas_call(
        functools.partial(_epilogue_kernel, wp=wp),
        out_shape=jax.ShapeDtypeStruct((n, cout, h2, w2s), jnp.float32),
        grid=(n,),
        in_specs=[
            pl.BlockSpec((1, cout, lf), lambda i: (i, 0, 0)),
            pl.BlockSpec((cout, 1), lambda i: (0, 0)),
            pl.BlockSpec((cout, 1), lambda i: (0, 0)),
        ],
        out_specs=pl.BlockSpec((1, cout, h2, w2s), lambda i: (i, 0, 0, 0)),
        compiler_params=pltpu.CompilerParams(
            dimension_semantics=("parallel",),
            vmem_limit_bytes=56 * 1024 * 1024),
        )(yk, scale, shift)
    return out


# speedup vs baseline: 1.6463x; 1.1374x over previous
import functools

import jax
import jax.numpy as jnp
from jax.experimental import pallas as pl
from jax.experimental.pallas import tpu as pltpu

_CD = jnp.bfloat16
_NT = 3


def _compiler_params():
    return pltpu.CompilerParams(
        dimension_semantics=("parallel",),
        vmem_limit_bytes=56 * 1024 * 1024)


def _up_vec(n, m):
    k = jnp.arange(m) // 2
    p = jnp.arange(m) % 2
    nb = jnp.where(p == 0, jnp.maximum(k - 1, 0), jnp.minimum(k + 1, n - 1))
    r = jnp.arange(n)[:, None]
    return 0.75 * (r == k[None, :]) + 0.25 * (r == nb[None, :])


def _upsample_matrix(h, w, wp):
    uv = _up_vec(h, 2 * h)
    uh = jnp.pad(_up_vec(w, 2 * w), ((0, 0), (1, 1)))
    b = uv[:, None, :, None] * uh[None, :, None, :]
    return b.reshape(h * w, 2 * h * wp)


def _stage_taps(a, stk9_ref, *, wp, xs):
    cin, lf = a.shape
    xfull = jnp.concatenate(
        [jnp.zeros((cin, wp + 1), _CD), a,
         jnp.zeros((cin, xs + 2 * wp + 2 - (wp + 1) - lf), _CD)], axis=1)
    for k in range(9):
        s = (k // 3) * wp + (k % 3)
        stk9_ref[k * cin:(k + 1) * cin, :] = xfull[:, s:s + xs]


def _conv_tiles(stk9_ref, w_ref, y_ref, ps_ref, pss_ref, *, wp, lt):
    ps = jnp.zeros((ps_ref.shape[1], 1), jnp.float32)
    pss = jnp.zeros((ps_ref.shape[1], 1), jnp.float32)
    for ti in range(_NT):
        f0 = ti * lt
        stk = stk9_ref[:, f0:f0 + lt]
        acc = jnp.dot(w_ref[...], stk, preferred_element_type=jnp.float32)
        y_ref[0, :, f0:f0 + lt] = acc.astype(y_ref.dtype)
        col = (jax.lax.broadcasted_iota(jnp.int32, (1, lt), 1) + f0) % wp
        am = jnp.where((col >= 1) & (col <= wp - 2), acc, 0.0)
        ps = ps + jnp.sum(am, axis=1, keepdims=True)
        pss = pss + jnp.sum(am * am, axis=1, keepdims=True)
    ps_ref[0] = ps
    pss_ref[0] = pss


def _conv1_kernel(x_ref, prev_ref, b_ref, w_ref, y_ref, ps_ref, pss_ref,
                  stk9_ref, *, wp, lt, xs):
    ax = jnp.dot(x_ref[0].astype(_CD), b_ref[...],
                 preferred_element_type=jnp.float32).astype(_CD)
    a = jnp.concatenate([ax, prev_ref[0]], axis=0)
    _stage_taps(a, stk9_ref, wp=wp, xs=xs)
    _conv_tiles(stk9_ref, w_ref, y_ref, ps_ref, pss_ref, wp=wp, lt=lt)


def _conv_mid_kernel(yin_ref, w_ref, sc_ref, sh_ref, y_ref, ps_ref, pss_ref,
                     stk9_ref, *, wp, lt, xs):
    lf = yin_ref.shape[2]
    v = yin_ref[0].astype(jnp.float32) * sc_ref[...] + sh_ref[...]
    v = jnp.maximum(v, 0.0)
    col = jax.lax.broadcasted_iota(jnp.int32, (1, lf), 1) % wp
    a = jnp.where((col >= 1) & (col <= wp - 2), v, 0.0).astype(_CD)
    _stage_taps(a, stk9_ref, wp=wp, xs=xs)
    _conv_tiles(stk9_ref, w_ref, y_ref, ps_ref, pss_ref, wp=wp, lt=lt)


def _epilogue_kernel(yin_ref, sc_ref, sh_ref, o_ref, *, wp):
    c, h2, w2 = o_ref.shape[1], o_ref.shape[2], o_ref.shape[3]
    v = jnp.maximum(yin_ref[0].astype(jnp.float32) * sc_ref[...] + sh_ref[...],
                    0.0)
    o_ref[0] = v.reshape(c, h2, wp)[:, :, 1:w2 + 1]


def _fold_bn(ps, pss, gamma, beta, count, eps=1e-5):
    cout = ps.shape[1]
    s = jnp.sum(ps.reshape(-1, cout), axis=0)
    ss = jnp.sum(pss.reshape(-1, cout), axis=0)
    mean = s / count
    var = jnp.maximum(ss / count - mean * mean, 0.0)
    scale = gamma / jnp.sqrt(var + eps)
    shift = beta - mean * scale
    return scale.reshape(-1, 1), shift.reshape(-1, 1)


def _stack_taps(w_oihw):
    cout, cin = w_oihw.shape[0], w_oihw.shape[1]
    return jnp.transpose(w_oihw, (0, 2, 3, 1)).reshape(cout, 9 * cin).astype(_CD)


def kernel(prev, x, w1, b1, gamma1, beta1, w2, b2, gamma2, beta2,
           w3, b3, gamma3, beta3):
    n, cx, h, w = x.shape
    cp = prev.shape[1]
    h2, w2s = 2 * h, 2 * w
    wp = w2s + 2
    lf = h2 * wp
    lt = lf // _NT
    count = n * h2 * w2s
    cin1 = cx + cp
    cout = w1.shape[0]
    xs = -(-lf // 128) * 128 + 256

    xf = x.reshape(n, cx, h * w)
    prevf = jnp.pad(prev, ((0, 0), (0, 0), (0, 0), (1, 1))
                    ).reshape(n, cp, lf).astype(_CD)
    bmat = _upsample_matrix(h, w, wp).astype(_CD)

    def conv_out_shape():
        return (jax.ShapeDtypeStruct((n, cout, lf), _CD),
                jax.ShapeDtypeStruct((n, cout, 1), jnp.float32),
                jax.ShapeDtypeStruct((n, cout, 1), jnp.float32))

    def conv_out_specs():
        return (pl.BlockSpec((1, cout, lf), lambda i: (i, 0, 0)),
                pl.BlockSpec((1, cout, 1), lambda i: (i, 0, 0)),
                pl.BlockSpec((1, cout, 1), lambda i: (i, 0, 0)))

    y1, ps, pss = pl.pallas_call(
        functools.partial(_conv1_kernel, wp=wp, lt=lt, xs=xs),
        out_shape=conv_out_shape(),
        grid=(n,),
        in_specs=[
            pl.BlockSpec((1, cx, h * w), lambda i: (i, 0, 0)),
            pl.BlockSpec((1, cp, lf), lambda i: (i, 0, 0)),
            pl.BlockSpec((h * w, lf), lambda i: (0, 0)),
            pl.BlockSpec((cout, 9 * cin1), lambda i: (0, 0)),
        ],
        out_specs=conv_out_specs(),
        scratch_shapes=[pltpu.VMEM((9 * cin1, xs), _CD)],
        compiler_params=_compiler_params(),
    )(xf, prevf, bmat, _stack_taps(w1))

    yk = y1
    scale, shift = _fold_bn(ps, pss, gamma1, beta1, count)
    for wi, gi, bi in ((w2, gamma2, beta2), (w3, gamma3, beta3)):
        ci = wi.shape[1]
        yk, ps, pss = pl.pallas_call(
            functools.partial(_conv_mid_kernel, wp=wp, lt=lt, xs=xs),
            out_shape=conv_out_shape(),
            grid=(n,),
            in_specs=[
                pl.BlockSpec((1, ci, lf), lambda i: (i, 0, 0)),
                pl.BlockSpec((cout, 9 * ci), lambda i: (0, 0)),
                pl.BlockSpec((ci, 1), lambda i: (0, 0)),
                pl.BlockSpec((ci, 1), lambda i: (0, 0)),
            ],
            out_specs=conv_out_specs(),
            scratch_shapes=[pltpu.VMEM((9 * ci, xs), _CD)],
            compiler_params=_compiler_params(),
        )(yk, _stack_taps(wi), scale, shift)
        scale, shift = _fold_bn(ps, pss, gi, bi, count)

    out = pl.pallas_call(
        functools.partial(_epilogue_kernel, wp=wp),
        out_shape=jax.ShapeDtypeStruct((n, cout, h2, w2s), jnp.float32),
        grid=(n,),
        in_specs=[
            pl.BlockSpec((1, cout, lf), lambda i: (i, 0, 0)),
            pl.BlockSpec((cout, 1), lambda i: (0, 0)),
            pl.BlockSpec((cout, 1), lambda i: (0, 0)),
        ],
        out_specs=pl.BlockSpec((1, cout, h2, w2s), lambda i: (i, 0, 0, 0)),
        compiler_params=_compiler_params(),
        )(yk, scale, shift)
    return out


# speedup vs baseline: 1.7039x; 1.0350x over previous
import functools

import jax
import jax.numpy as jnp
import numpy as np
from jax.experimental import pallas as pl
from jax.experimental.pallas import tpu as pltpu

_CD = jnp.bfloat16
_NT = 3


def _compiler_params():
    return pltpu.CompilerParams(
        dimension_semantics=("parallel",),
        vmem_limit_bytes=56 * 1024 * 1024)


def _up_vec(n, m):
    k = np.arange(m) // 2
    p = np.arange(m) % 2
    nb = np.where(p == 0, np.maximum(k - 1, 0), np.minimum(k + 1, n - 1))
    r = np.arange(n)[:, None]
    return 0.75 * (r == k[None, :]) + 0.25 * (r == nb[None, :])


def _upsample_matrix(h, w, wp):
    uv = _up_vec(h, 2 * h)
    uh = np.pad(_up_vec(w, 2 * w), ((0, 0), (1, 1)))
    b = uv[:, None, :, None] * uh[None, :, None, :]
    return jnp.asarray(b.reshape(h * w, 2 * h * wp), dtype=_CD)


def _stage_taps(a, stk9_ref, *, wp, xs):
    cin, lf = a.shape
    xfull = jnp.concatenate(
        [jnp.zeros((cin, wp + 1), _CD), a,
         jnp.zeros((cin, xs + 2 * wp + 2 - (wp + 1) - lf), _CD)], axis=1)
    for k in range(9):
        s = (k // 3) * wp + (k % 3)
        stk9_ref[k * cin:(k + 1) * cin, :] = xfull[:, s:s + xs]


def _conv_tiles(stk9_ref, w_ref, y_ref, ps_ref, pss_ref, *, wp, lt):
    ps = jnp.zeros((ps_ref.shape[1], 1), jnp.float32)
    pss = jnp.zeros((ps_ref.shape[1], 1), jnp.float32)
    for ti in range(_NT):
        f0 = ti * lt
        stk = stk9_ref[:, f0:f0 + lt]
        acc = jnp.dot(w_ref[...], stk, preferred_element_type=jnp.float32)
        y_ref[0, :, f0:f0 + lt] = acc.astype(y_ref.dtype)
        col = (jax.lax.broadcasted_iota(jnp.int32, (1, lt), 1) + f0) % wp
        am = jnp.where((col >= 1) & (col <= wp - 2), acc, 0.0)
        ps = ps + jnp.sum(am, axis=1, keepdims=True)
        pss = pss + jnp.sum(am * am, axis=1, keepdims=True)
    ps_ref[0] = ps
    pss_ref[0] = pss


def _conv1_kernel(x_ref, prev_ref, b_ref, w_ref, y_ref, ps_ref, pss_ref,
                  stk9_ref, *, wp, lt, xs):
    ax = jnp.dot(x_ref[0].astype(_CD), b_ref[...],
                 preferred_element_type=jnp.float32).astype(_CD)
    pv = prev_ref[0].astype(_CD)
    cp, h2 = pv.shape[0], pv.shape[1]
    zc = jnp.zeros((cp, h2, 1), _CD)
    ap = jnp.concatenate([zc, pv, zc], axis=2).reshape(cp, h2 * wp)
    a = jnp.concatenate([ax, ap], axis=0)
    _stage_taps(a, stk9_ref, wp=wp, xs=xs)
    _conv_tiles(stk9_ref, w_ref, y_ref, ps_ref, pss_ref, wp=wp, lt=lt)


def _conv_mid_kernel(yin_ref, w_ref, sc_ref, sh_ref, y_ref, ps_ref, pss_ref,
                     stk9_ref, *, wp, lt, xs):
    lf = yin_ref.shape[2]
    v = yin_ref[0].astype(jnp.float32) * sc_ref[...] + sh_ref[...]
    v = jnp.maximum(v, 0.0)
    col = jax.lax.broadcasted_iota(jnp.int32, (1, lf), 1) % wp
    a = jnp.where((col >= 1) & (col <= wp - 2), v, 0.0).astype(_CD)
    _stage_taps(a, stk9_ref, wp=wp, xs=xs)
    _conv_tiles(stk9_ref, w_ref, y_ref, ps_ref, pss_ref, wp=wp, lt=lt)


def _epilogue_kernel(yin_ref, sc_ref, sh_ref, o_ref, *, wp):
    c, h2, w2 = o_ref.shape[1], o_ref.shape[2], o_ref.shape[3]
    v = jnp.maximum(yin_ref[0].astype(jnp.float32) * sc_ref[...] + sh_ref[...],
                    0.0)
    o_ref[0] = v.reshape(c, h2, wp)[:, :, 1:w2 + 1]


def _fold_bn(ps, pss, gamma, beta, count, eps=1e-5):
    cout = ps.shape[1]
    s = jnp.sum(ps.reshape(-1, cout), axis=0)
    ss = jnp.sum(pss.reshape(-1, cout), axis=0)
    mean = s / count
    var = jnp.maximum(ss / count - mean * mean, 0.0)
    scale = gamma / jnp.sqrt(var + eps)
    shift = beta - mean * scale
    return scale.reshape(-1, 1), shift.reshape(-1, 1)


def _stack_taps(w_oihw):
    cout, cin = w_oihw.shape[0], w_oihw.shape[1]
    return jnp.transpose(w_oihw, (0, 2, 3, 1)).reshape(cout, 9 * cin).astype(_CD)


def kernel(prev, x, w1, b1, gamma1, beta1, w2, b2, gamma2, beta2,
           w3, b3, gamma3, beta3):
    n, cx, h, w = x.shape
    cp = prev.shape[1]
    h2, w2s = 2 * h, 2 * w
    wp = w2s + 2
    lf = h2 * wp
    lt = lf // _NT
    count = n * h2 * w2s
    cin1 = cx + cp
    cout = w1.shape[0]
    xs = -(-lf // 128) * 128 + 256

    xf = x.reshape(n, cx, h * w)
    bmat = _upsample_matrix(h, w, wp)

    def conv_out_shape():
        return (jax.ShapeDtypeStruct((n, cout, lf), _CD),
                jax.ShapeDtypeStruct((n, cout, 1), jnp.float32),
                jax.ShapeDtypeStruct((n, cout, 1), jnp.float32))

    def conv_out_specs():
        return (pl.BlockSpec((1, cout, lf), lambda i: (i, 0, 0)),
                pl.BlockSpec((1, cout, 1), lambda i: (i, 0, 0)),
                pl.BlockSpec((1, cout, 1), lambda i: (i, 0, 0)))

    y1, ps, pss = pl.pallas_call(
        functools.partial(_conv1_kernel, wp=wp, lt=lt, xs=xs),
        out_shape=conv_out_shape(),
        grid=(n,),
        in_specs=[
            pl.BlockSpec((1, cx, h * w), lambda i: (i, 0, 0)),
            pl.BlockSpec((1, cp, h2, w2s), lambda i: (i, 0, 0, 0)),
            pl.BlockSpec((h * w, lf), lambda i: (0, 0)),
            pl.BlockSpec((cout, 9 * cin1), lambda i: (0, 0)),
        ],
        out_specs=conv_out_specs(),
        scratch_shapes=[pltpu.VMEM((9 * cin1, xs), _CD)],
        compiler_params=_compiler_params(),
    )(xf, prev, bmat, _stack_taps(w1))

    yk = y1
    scale, shift = _fold_bn(ps, pss, gamma1, beta1, count)
    for wi, gi, bi in ((w2, gamma2, beta2), (w3, gamma3, beta3)):
        ci = wi.shape[1]
        yk, ps, pss = pl.pallas_call(
            functools.partial(_conv_mid_kernel, wp=wp, lt=lt, xs=xs),
            out_shape=conv_out_shape(),
            grid=(n,),
            in_specs=[
                pl.BlockSpec((1, ci, lf), lambda i: (i, 0, 0)),
                pl.BlockSpec((cout, 9 * ci), lambda i: (0, 0)),
                pl.BlockSpec((ci, 1), lambda i: (0, 0)),
                pl.BlockSpec((ci, 1), lambda i: (0, 0)),
            ],
            out_specs=conv_out_specs(),
            scratch_shapes=[pltpu.VMEM((9 * ci, xs), _CD)],
            compiler_params=_compiler_params(),
        )(yk, _stack_taps(wi), scale, shift)
        scale, shift = _fold_bn(ps, pss, gi, bi, count)

    out = pl.pallas_call(
        functools.partial(_epilogue_kernel, wp=wp),
        out_shape=jax.ShapeDtypeStruct((n, cout, h2, w2s), jnp.float32),
        grid=(n,),
        in_specs=[
            pl.BlockSpec((1, cout, lf), lambda i: (i, 0, 0)),
            pl.BlockSpec((cout, 1), lambda i: (0, 0)),
            pl.BlockSpec((cout, 1), lambda i: (0, 0)),
        ],
        out_specs=pl.BlockSpec((1, cout, h2, w2s), lambda i: (i, 0, 0, 0)),
        compiler_params=_compiler_params(),
        )(yk, scale, shift)
    return out
